# Initial kernel scaffold; baseline (speedup 1.0000x reference)
import jax
import jax.numpy as jnp
from jax.experimental import pallas as pl


def kernel(edge_index, edge_type, dynamic_emb, emb_rel, W_layers, w1, w2):
    raise NotImplementedError("write your pallas kernel here")


# speedup vs baseline: 3.3847x; 3.3847x over previous
import functools

import jax
import jax.numpy as jnp
from jax import lax
from jax.experimental import pallas as pl
from jax.experimental.pallas import tpu as pltpu
from jax.experimental.pallas import tpu_sc as plsc

N = 10000
E = 320000
D = 128
NREL = 200
SQRT_C = 0.1
SLOPE = (1.0 / 8.0 + 1.0 / 3.0) / 2.0

NC = 2
NS = 16
NW = NC * NS
EPW = E // NW
CH = 80
NCHUNK = EPW // CH
NPAD = 10112
RPT = NPAD // NS

NN = 10240
HALF = NN // 2
RPT_E = HALF // NS
EPT = E // NS
NCHUNK_E = EPT // CH

BLK = 1000
GRID = N // BLK


def _expmap0(x):
    n = jnp.sqrt(jnp.sum(x * x, axis=-1, keepdims=True))
    n = jnp.clip(n, 1e-10, None)
    return jnp.tanh(SQRT_C * n) * x / (SQRT_C * n)


def _logmap0(p):
    n = jnp.sqrt(jnp.sum(p * p, axis=-1, keepdims=True))
    n = jnp.clip(n, 1e-10, None)
    a = jnp.clip(SQRT_C * n, 0.0, 1.0 - 1e-5)
    atanh = 0.5 * jnp.log((1.0 + a) / (1.0 - a))
    return atanh * p / (SQRT_C * n)


def _leaky(a):
    return jnp.where(a >= 0, a, a * SLOPE)


def _prep_body(u_ref, w_ref, er_ref, hw_ref, ht_ref, rw_ref):
    ht = _logmap0(_expmap0(u_ref[...]))
    ht_ref[...] = ht
    hw_ref[...] = jnp.dot(ht, w_ref[...], preferred_element_type=jnp.float32)

    @pl.when(pl.program_id(0) == 0)
    def _():
        rw_ref[...] = jnp.dot(er_ref[...], w_ref[...],
                              preferred_element_type=jnp.float32)


def _tc_prep(u, w, er):
    return pl.pallas_call(
        _prep_body,
        grid=(GRID,),
        in_specs=[
            pl.BlockSpec((BLK, D), lambda i: (i, 0)),
            pl.BlockSpec((D, D), lambda i: (0, 0)),
            pl.BlockSpec((NREL, D), lambda i: (0, 0)),
        ],
        out_specs=[
            pl.BlockSpec((BLK, D), lambda i: (i, 0)),
            pl.BlockSpec((BLK, D), lambda i: (i, 0)),
            pl.BlockSpec((NREL, D), lambda i: (0, 0)),
        ],
        out_shape=[
            jax.ShapeDtypeStruct((N, D), jnp.float32),
            jax.ShapeDtypeStruct((N, D), jnp.float32),
            jax.ShapeDtypeStruct((NREL, D), jnp.float32),
        ],
    )(u, w, er)


def _combine_body(acc_ref, deg_ref, htp_ref, w_ref, er_ref,
                  hw_ref, ht_ref, rw_ref):
    d = deg_ref[:, 0:1]
    a = acc_ref[...] / jnp.clip(d, 1.0, None) + htp_ref[...]
    ht = _logmap0(_expmap0(_leaky(a)))
    ht_ref[...] = ht
    hw_ref[...] = jnp.dot(ht, w_ref[...], preferred_element_type=jnp.float32)

    @pl.when(pl.program_id(0) == 0)
    def _():
        rw_ref[...] = jnp.dot(er_ref[...], w_ref[...],
                              preferred_element_type=jnp.float32)


def _tc_combine(acc, deg, htp, w, er):
    return pl.pallas_call(
        _combine_body,
        grid=(GRID,),
        in_specs=[
            pl.BlockSpec((BLK, D), lambda i: (i, 0)),
            pl.BlockSpec((BLK, D), lambda i: (i, 0)),
            pl.BlockSpec((BLK, D), lambda i: (i, 0)),
            pl.BlockSpec((D, D), lambda i: (0, 0)),
            pl.BlockSpec((NREL, D), lambda i: (0, 0)),
        ],
        out_specs=[
            pl.BlockSpec((BLK, D), lambda i: (i, 0)),
            pl.BlockSpec((BLK, D), lambda i: (i, 0)),
            pl.BlockSpec((NREL, D), lambda i: (0, 0)),
        ],
        out_shape=[
            jax.ShapeDtypeStruct((N, D), jnp.float32),
            jax.ShapeDtypeStruct((N, D), jnp.float32),
            jax.ShapeDtypeStruct((NREL, D), jnp.float32),
        ],
    )(acc, deg, htp, w, er)


def _final_body(acc_ref, deg_ref, htp_ref, u_ref, w1_ref, w2_ref, out_ref):
    d = deg_ref[:, 0:1]
    a = acc_ref[...] / jnp.clip(d, 1.0, None) + htp_ref[...]
    ht = _logmap0(_expmap0(_leaky(a)))
    pt = _logmap0(_expmap0(u_ref[...]))
    gate = jax.nn.sigmoid(
        jnp.dot(ht, w1_ref[...], preferred_element_type=jnp.float32)
        + jnp.dot(pt, w2_ref[...], preferred_element_type=jnp.float32))
    out_ref[...] = _expmap0(gate * ht + (1.0 - gate) * pt)


def _tc_final(acc, deg, htp, u, w1, w2):
    return pl.pallas_call(
        _final_body,
        grid=(GRID,),
        in_specs=[
            pl.BlockSpec((BLK, D), lambda i: (i, 0)),
            pl.BlockSpec((BLK, D), lambda i: (i, 0)),
            pl.BlockSpec((BLK, D), lambda i: (i, 0)),
            pl.BlockSpec((BLK, D), lambda i: (i, 0)),
            pl.BlockSpec((D, D), lambda i: (0, 0)),
            pl.BlockSpec((D, D), lambda i: (0, 0)),
        ],
        out_specs=pl.BlockSpec((BLK, D), lambda i: (i, 0)),
        out_shape=jax.ShapeDtypeStruct((N, D), jnp.float32),
    )(acc, deg, htp, u, w1, w2)


def _sc_edge_body(idx3_h, hw_h, rw_h, out_h,
                  acc_s, idx3a, idx3b, didx2, hrowsa, hrowsb, rrowsa, rrowsb,
                  zbuf, semha, semhb, semra, semrb):
    cid = lax.axis_index("c")
    sid = lax.axis_index("s")
    base = sid * RPT_E
    lo_node = cid * HALF
    ebase = sid * NCHUNK_E

    idx3 = [idx3a, idx3b]
    hrows = [hrowsa, hrowsb]
    rrows = [rrowsa, rrowsb]
    semh = [semha, semhb]
    semr = [semra, semrb]

    zeros16 = jnp.zeros((16,), jnp.float32)
    lane8 = jnp.arange(16, dtype=jnp.int32) & 7

    @pl.loop(0, RPT_E)
    def _(i):
        for j in range(D // 16):
            zbuf[i, pl.ds(j * 16, 16)] = zeros16

    pltpu.sync_copy(zbuf, acc_s.at[pl.ds(base, RPT_E)])

    plsc.subcore_barrier()

    def issue(t, s):
        off = pl.multiple_of((ebase + t) * (3 * CH), 8)
        pltpu.sync_copy(idx3_h.at[pl.ds(off, 3 * CH)], idx3[s])
        pltpu.async_copy(hw_h.at[idx3[s].at[pl.ds(0, CH)]], hrows[s], semh[s])
        pltpu.async_copy(rw_h.at[idx3[s].at[pl.ds(CH, CH)]], rrows[s],
                         semr[s])

    issue(0, 0)

    @pl.loop(0, NCHUNK_E, step=2)
    def _(t):
        for b in range(2):
            tt = t + b
            s = b

            @pl.when(tt + 1 < NCHUNK_E)
            def _():
                issue(tt + 1, 1 - b)

            pltpu.make_async_copy(hw_h.at[idx3[s].at[pl.ds(0, CH)]],
                                  hrows[s], semh[s]).wait()
            pltpu.make_async_copy(rw_h.at[idx3[s].at[pl.ds(CH, CH)]],
                                  rrows[s], semr[s]).wait()

            for k in range(CH // 16):
                v = idx3[s][pl.ds(2 * CH + k * 16, 16)] - lo_node
                ok = (v >= 0) & (v < HALF)
                didx2[pl.ds(k * 16, 16)] = jnp.where(ok, v, HALF + lane8)

            pltpu.sync_copy(hrows[s], acc_s.at[didx2], add=True)
            pltpu.sync_copy(rrows[s], acc_s.at[didx2], add=True)

    plsc.subcore_barrier()
    pltpu.sync_copy(acc_s.at[pl.ds(base, RPT_E)],
                    out_h.at[cid].at[pl.ds(base, RPT_E)])


_sc_edge = pl.kernel(
    _sc_edge_body,
    out_type=jax.ShapeDtypeStruct((NC, HALF, D), jnp.float32),
    mesh=plsc.VectorSubcoreMesh(core_axis_name="c", subcore_axis_name="s"),
    scratch_types=[
        pltpu.VMEM_SHARED((HALF + 8, D), jnp.float32),
        pltpu.VMEM((3 * CH,), jnp.int32),
        pltpu.VMEM((3 * CH,), jnp.int32),
        pltpu.VMEM((CH,), jnp.int32),
        pltpu.VMEM((CH, D), jnp.float32),
        pltpu.VMEM((CH, D), jnp.float32),
        pltpu.VMEM((CH, D), jnp.float32),
        pltpu.VMEM((CH, D), jnp.float32),
        pltpu.VMEM((RPT_E, D), jnp.float32),
        pltpu.SemaphoreType.DMA,
        pltpu.SemaphoreType.DMA,
        pltpu.SemaphoreType.DMA,
        pltpu.SemaphoreType.DMA,
    ],
)


def _sc_deg_body(dst_h, out_h, deg_s, didx, didx2, ones, zbuf):
    cid = lax.axis_index("c")
    sid = lax.axis_index("s")
    base = sid * RPT_E
    lo_node = cid * HALF

    zeros16 = jnp.zeros((16,), jnp.float32)
    ones16 = jnp.full((16,), 1.0, jnp.float32)

    @pl.loop(0, RPT_E)
    def _(i):
        for j in range(D // 16):
            zbuf[i, pl.ds(j * 16, 16)] = zeros16

    @pl.loop(0, CH)
    def _(i):
        for j in range(D // 16):
            ones[i, pl.ds(j * 16, 16)] = ones16

    pltpu.sync_copy(zbuf, deg_s.at[pl.ds(base, RPT_E)])

    plsc.subcore_barrier()

    @pl.loop(0, NCHUNK_E)
    def _(t):
        off = pl.multiple_of(sid * EPT + t * CH, 8)
        pltpu.sync_copy(dst_h.at[pl.ds(off, CH)], didx)
        for k in range(CH // 16):
            v = didx[pl.ds(k * 16, 16)] - lo_node
            ok = (v >= 0) & (v < HALF)
            didx2[pl.ds(k * 16, 16)] = jnp.where(ok, v, HALF)
        pltpu.sync_copy(ones, deg_s.at[didx2], add=True)

    plsc.subcore_barrier()
    pltpu.sync_copy(deg_s.at[pl.ds(base, RPT_E)],
                    out_h.at[cid].at[pl.ds(base, RPT_E)])


_sc_deg = pl.kernel(
    _sc_deg_body,
    out_type=jax.ShapeDtypeStruct((NC, HALF, D), jnp.float32),
    mesh=plsc.VectorSubcoreMesh(core_axis_name="c", subcore_axis_name="s"),
    scratch_types=[
        pltpu.VMEM_SHARED((HALF + 8, D), jnp.float32),
        pltpu.VMEM((CH,), jnp.int32),
        pltpu.VMEM((CH,), jnp.int32),
        pltpu.VMEM((CH, D), jnp.float32),
        pltpu.VMEM((RPT_E, D), jnp.float32),
    ],
)


@jax.jit
def kernel(edge_index, edge_type, dynamic_emb, emb_rel, W_layers, w1, w2):
    src = edge_index[0].astype(jnp.int32)
    dst = edge_index[1].astype(jnp.int32)
    et = edge_type.astype(jnp.int32)
    edges3 = jnp.stack([src.reshape(-1, CH), et.reshape(-1, CH),
                        dst.reshape(-1, CH)], axis=1).reshape(-1)

    deg = _sc_deg(dst).reshape(NN, D)[:N]

    hw1, ht1, rw1 = _tc_prep(dynamic_emb, W_layers[0], emb_rel)
    acc1 = _sc_edge(edges3, hw1, rw1).reshape(NN, D)[:N]
    hw2, ht2, rw2 = _tc_combine(acc1, deg, ht1, W_layers[1], emb_rel)
    acc2 = _sc_edge(edges3, hw2, rw2).reshape(NN, D)[:N]
    return _tc_final(acc2, deg, ht2, dynamic_emb, w1, w2)
